# baseline (speedup 1.0000x reference)
import functools

import jax
import jax.numpy as jnp
from jax import lax
from jax.experimental import pallas as pl
from jax.experimental.pallas import tpu as pltpu


def _sepconv_kernel(x_ref, wd_ref, wp_ref, o_ref, wm_ref, *, H, W, KH, KW,
                    dilation, padding, imgs, split):
    HW = H * W
    C = x_ref.shape[1]

    @pl.when((pl.program_id(0) == 0) & (pl.program_id(1) == 0))
    def _build_weight_planes():
        p_idx = lax.broadcasted_iota(jnp.int32, (HW, C), 0)
        hh = p_idx // W
        ww = p_idx - hh * W
        for kh in range(KH):
            dh = kh * dilation - padding
            for kw in range(KW):
                dw = kw * dilation - padding
                t = kh * KW + kw
                wrow = wd_ref[t:t + 1, :].astype(jnp.bfloat16)
                plane = jnp.broadcast_to(wrow, (HW, C))
                for cond in (
                        (hh >= -dh) if dh < 0 else None,
                        (hh < H - dh) if dh > 0 else None,
                        (ww >= -dw) if dw < 0 else None,
                        (ww < W - dw) if dw > 0 else None):
                    if cond is not None:
                        plane = plane * cond.astype(jnp.bfloat16)
                wm_ref[t * HW:(t + 1) * HW, :] = plane

    wp = wp_ref[...].astype(jnp.bfloat16)
    half = imgs // split
    j = pl.program_id(1)
    for i in range(half):
        xi = j * half + i
        xb = x_ref.at[pl.ds(xi * HW, HW), :][...].astype(jnp.bfloat16)
        acc = None
        for kh in range(KH):
            dh = kh * dilation - padding
            for kw in range(KW):
                dw = kw * dilation - padding
                t = kh * KW + kw
                shift = dh * W + dw
                if shift == 0:
                    patch = xb
                else:
                    patch = pltpu.roll(xb, shift=(-shift) % HW, axis=0)
                term = patch * wm_ref[t * HW:(t + 1) * HW, :]
                acc = term if acc is None else acc + term

        out = jnp.dot(acc, wp, preferred_element_type=jnp.float32)
        o_ref[i * HW:(i + 1) * HW, :] = out.astype(o_ref.dtype)
        del out


def kernel(x_nchw, w_dw, w_pw):
    N, C, H, W = x_nchw.shape
    KH, KW, _ = w_dw.shape
    O = w_pw.shape[1]
    HW = H * W
    dilation, padding = 1, 1

    x2 = jnp.transpose(x_nchw, (0, 2, 3, 1)).reshape(N * HW, C)
    wd = w_dw.reshape(KH * KW, C)

    imgs = 8 if N % 8 == 0 else 1
    split = 2 if imgs % 2 == 0 else 1
    kernel_fn = functools.partial(_sepconv_kernel, H=H, W=W, KH=KH, KW=KW,
                                  dilation=dilation, padding=padding,
                                  imgs=imgs, split=split)

    out2 = pl.pallas_call(
        kernel_fn,
        out_shape=jax.ShapeDtypeStruct((N * HW, O), x_nchw.dtype),
        grid_spec=pltpu.PrefetchScalarGridSpec(
            num_scalar_prefetch=0,
            grid=(N // imgs, split),
            in_specs=[
                pl.BlockSpec((imgs * HW, C), lambda g, j: (g, 0)),
                pl.BlockSpec((KH * KW, C), lambda g, j: (0, 0)),
                pl.BlockSpec((C, O), lambda g, j: (0, 0)),
            ],
            out_specs=pl.BlockSpec((imgs * HW // split, O),
                                   lambda g, j: (split * g + j, 0)),
            scratch_shapes=[pltpu.VMEM((KH * KW * HW, C), jnp.bfloat16)],
        ),
        compiler_params=pltpu.CompilerParams(
            dimension_semantics=("arbitrary", "arbitrary"),
            vmem_limit_bytes=56 << 20),
    )(x2, wd, w_pw)

    return out2.reshape(N, H, W, O).transpose(0, 3, 1, 2)


# speedup vs baseline: 1.4201x; 1.4201x over previous
import functools

import jax
import jax.numpy as jnp
from jax import lax
from jax.experimental import pallas as pl
from jax.experimental.pallas import tpu as pltpu


def _sepconv_kernel(x_ref, wd_ref, wp_ref, o_ref, wm_ref, *, H, W, KH, KW,
                    dilation, padding, imgs):
    HW = H * W
    C = x_ref.shape[1]

    @pl.when(pl.program_id(0) == 0)
    def _build_weight_planes():
        p_idx = lax.broadcasted_iota(jnp.int32, (HW, C), 0)
        hh = p_idx // W
        ww = p_idx - hh * W
        for kh in range(KH):
            dh = kh * dilation - padding
            for kw in range(KW):
                dw = kw * dilation - padding
                t = kh * KW + kw
                wrow = wd_ref[t:t + 1, :].astype(jnp.bfloat16)
                plane = jnp.broadcast_to(wrow, (HW, C))
                for cond in (
                        (hh >= -dh) if dh < 0 else None,
                        (hh < H - dh) if dh > 0 else None,
                        (ww >= -dw) if dw < 0 else None,
                        (ww < W - dw) if dw > 0 else None):
                    if cond is not None:
                        plane = plane * cond.astype(jnp.bfloat16)
                wm_ref[t * HW:(t + 1) * HW, :] = plane

    wp = wp_ref[...].astype(jnp.bfloat16)
    for i in range(imgs):
        xb = x_ref[i * HW:(i + 1) * HW, :].astype(jnp.bfloat16)
        acc = None
        for kh in range(KH):
            dh = kh * dilation - padding
            for kw in range(KW):
                dw = kw * dilation - padding
                t = kh * KW + kw
                shift = dh * W + dw
                if shift == 0:
                    patch = xb
                else:
                    patch = pltpu.roll(xb, shift=(-shift) % HW, axis=0)
                term = patch * wm_ref[t * HW:(t + 1) * HW, :]
                acc = term if acc is None else acc + term

        out = jnp.dot(acc, wp, preferred_element_type=jnp.float32)
        o_ref[i * HW:(i + 1) * HW, :] = out.astype(o_ref.dtype)


def kernel(x_nchw, w_dw, w_pw):
    N, C, H, W = x_nchw.shape
    KH, KW, _ = w_dw.shape
    O = w_pw.shape[1]
    HW = H * W
    dilation, padding = 1, 1

    x2 = jnp.transpose(x_nchw, (0, 2, 3, 1)).reshape(N * HW, C)
    wd = w_dw.reshape(KH * KW, C)

    imgs = 8 if N % 8 == 0 else 1
    kernel_fn = functools.partial(_sepconv_kernel, H=H, W=W, KH=KH, KW=KW,
                                  dilation=dilation, padding=padding,
                                  imgs=imgs)

    out2 = pl.pallas_call(
        kernel_fn,
        out_shape=jax.ShapeDtypeStruct((N * HW, O), x_nchw.dtype),
        grid_spec=pltpu.PrefetchScalarGridSpec(
            num_scalar_prefetch=0,
            grid=(N // imgs,),
            in_specs=[
                pl.BlockSpec((imgs * HW, C), lambda g: (g, 0)),
                pl.BlockSpec((KH * KW, C), lambda g: (0, 0)),
                pl.BlockSpec((C, O), lambda g: (0, 0)),
            ],
            out_specs=pl.BlockSpec((imgs * HW, O), lambda g: (g, 0)),
            scratch_shapes=[pltpu.VMEM((KH * KW * HW, C), jnp.bfloat16)],
        ),
        compiler_params=pltpu.CompilerParams(
            dimension_semantics=("arbitrary",),
            vmem_limit_bytes=56 << 20),
    )(x2, wd, w_pw)

    return out2.reshape(N, H, W, O).transpose(0, 3, 1, 2)


# speedup vs baseline: 1.4281x; 1.0057x over previous
import functools

import jax
import jax.numpy as jnp
from jax import lax
from jax.experimental import pallas as pl
from jax.experimental.pallas import tpu as pltpu


def _sepconv_kernel(x_ref, wd_ref, wp_ref, o_ref, wm_ref, *, H, W, KH, KW,
                    dilation, padding, imgs):
    HW = H * W
    C = x_ref.shape[1]

    @pl.when(pl.program_id(0) == 0)
    def _build_weight_planes():
        p_idx = lax.broadcasted_iota(jnp.int32, (HW, C), 0)
        hh = p_idx // W
        ww = p_idx - hh * W
        for kh in range(KH):
            dh = kh * dilation - padding
            for kw in range(KW):
                dw = kw * dilation - padding
                t = kh * KW + kw
                wrow = wd_ref[t:t + 1, :].astype(jnp.bfloat16)
                plane = jnp.broadcast_to(wrow, (HW, C))
                for cond in (
                        (hh >= -dh) if dh < 0 else None,
                        (hh < H - dh) if dh > 0 else None,
                        (ww >= -dw) if dw < 0 else None,
                        (ww < W - dw) if dw > 0 else None):
                    if cond is not None:
                        plane = plane * cond.astype(jnp.bfloat16)
                wm_ref[t * HW:(t + 1) * HW, :] = plane

    wp = wp_ref[...].astype(jnp.bfloat16)
    for i in range(imgs):
        xb = x_ref[i * HW:(i + 1) * HW, :].astype(jnp.bfloat16)
        acc = None
        for kw in range(KW):
            dw = kw * dilation - padding
            if dw == 0:
                z = xb
            else:
                z = pltpu.roll(xb, shift=(-dw) % HW, axis=0)
            for kh in range(KH):
                dh = kh * dilation - padding
                t = kh * KW + kw
                if dh == 0:
                    patch = z
                else:
                    patch = pltpu.roll(z, shift=(-dh * W) % HW, axis=0)
                term = patch * wm_ref[t * HW:(t + 1) * HW, :]
                acc = term if acc is None else acc + term

        out = jnp.dot(acc, wp, preferred_element_type=jnp.float32)
        o_ref[i * HW:(i + 1) * HW, :] = out.astype(o_ref.dtype)


def kernel(x_nchw, w_dw, w_pw):
    N, C, H, W = x_nchw.shape
    KH, KW, _ = w_dw.shape
    O = w_pw.shape[1]
    HW = H * W
    dilation, padding = 1, 1

    x2 = jnp.transpose(x_nchw, (0, 2, 3, 1)).reshape(N * HW, C)
    wd = w_dw.reshape(KH * KW, C)

    imgs = 8 if N % 8 == 0 else 1
    kernel_fn = functools.partial(_sepconv_kernel, H=H, W=W, KH=KH, KW=KW,
                                  dilation=dilation, padding=padding,
                                  imgs=imgs)

    out2 = pl.pallas_call(
        kernel_fn,
        out_shape=jax.ShapeDtypeStruct((N * HW, O), x_nchw.dtype),
        grid_spec=pltpu.PrefetchScalarGridSpec(
            num_scalar_prefetch=0,
            grid=(N // imgs,),
            in_specs=[
                pl.BlockSpec((imgs * HW, C), lambda g: (g, 0)),
                pl.BlockSpec((KH * KW, C), lambda g: (0, 0)),
                pl.BlockSpec((C, O), lambda g: (0, 0)),
            ],
            out_specs=pl.BlockSpec((imgs * HW, O), lambda g: (g, 0)),
            scratch_shapes=[pltpu.VMEM((KH * KW * HW, C), jnp.bfloat16)],
        ),
        compiler_params=pltpu.CompilerParams(
            dimension_semantics=("arbitrary",),
            vmem_limit_bytes=56 << 20),
    )(x2, wd, w_pw)

    return out2.reshape(N, H, W, O).transpose(0, 3, 1, 2)
